# baseline (speedup 1.0000x reference)
import functools

import jax
import jax.numpy as jnp
from jax import lax
from jax.experimental import pallas as pl
from jax.experimental.pallas import tpu as pltpu
from jax.experimental.pallas import tpu_sc as plsc

_B, _S, _H = 16, 512, 512
_T = 2048
_L = 16
_WPB = 2
_TPW = _T // _WPB
_C = 64
_NCH = _TPW // _C


def _round_half_even(v):
    ti = v.astype(jnp.int32)
    frac = v - ti.astype(jnp.float32)
    half = jnp.full((_L,), 0.5, jnp.float32)
    one = jnp.ones((_L,), jnp.int32)
    zero = jnp.zeros((_L,), jnp.int32)
    up = jnp.where(frac > half, one, zero)
    tie = jnp.where(frac == half, one, zero) & (ti & one)
    return ti + (up | tie)


def _expand_body(x_hbm, dur_hbm, ml_hbm, out_hbm,
                 dur_v, cum_v, idx_v, ml_v,
                 rows0_v, rows1_v, rows2_v,
                 gsem0, gsem1, gsem2, osem0, osem1, osem2):
    half = lax.axis_index("c")
    b = lax.axis_index("s")
    t0 = half * _TPW
    row0 = b * _T + t0

    pltpu.sync_copy(dur_hbm.at[pl.ds(b * _S, _S)], dur_v)
    pltpu.sync_copy(ml_hbm, ml_v)
    max_len = ml_v[...][0]

    def cs_body(i, carry):
        r = _round_half_even(dur_v[pl.ds(i * _L, _L)])
        cum = plsc.cumsum(r) + carry
        cum_v[pl.ds(i * _L, _L)] = cum
        return cum[_L - 1]

    total = lax.fori_loop(0, _S // _L, cs_body, jnp.int32(0))

    def ss_body(i, _):
        t = t0 + i * _L + lax.iota(jnp.int32, _L)
        base = jnp.zeros((_L,), jnp.int32)
        for step in (256, 128, 64, 32, 16, 8, 4, 2, 1):
            probe = base + (step - 1)
            g = plsc.load_gather(cum_v, [probe])
            base = jnp.where(g <= t, base + step, base)
        idx_v[pl.ds(i * _L, _L)] = base + b * _S
        return 0

    _PRE = 2 * _C // _L
    lax.fori_loop(0, _PRE, ss_body, 0)

    nv = jnp.clip(jnp.minimum(total, max_len) - t0, 0, _TPW)

    zvec = jnp.zeros((_L,), jnp.float32)
    bufs = (rows0_v, rows1_v, rows2_v)
    gsems = (gsem0, gsem1, gsem2)
    osems = (osem0, osem1, osem2)

    def chunk_k(c):
        return jnp.clip(nv - c * _C, 0, _C)

    def start_gather(c):
        @pl.when(chunk_k(c) > 0)
        def _():
            pltpu.async_copy(
                x_hbm.at[idx_v.at[pl.ds(c * _C, _C)]],
                bufs[c % 3], gsems[c % 3])

    def wait_gather(c):
        @pl.when(chunk_k(c) > 0)
        def _():
            pltpu.make_async_copy(
                x_hbm.at[pl.ds(0, _C)], bufs[c % 3], gsems[c % 3]).wait()

    out_handles = {}
    nz = [jnp.int32(_C)] * 3

    start_gather(0)
    start_gather(1)
    lax.fori_loop(_PRE, _TPW // _L, ss_body, 0)
    for c in range(_NCH):
        if c >= 1:
            out_handles.pop(c - 1).wait()
        if c + 2 < _NCH:
            start_gather(c + 2)
        wait_gather(c)
        k = chunk_k(c)
        buf = bufs[c % 3]
        lo = jnp.where(k > 0, k, jnp.int32(0))
        hi = jnp.where(k > 0, jnp.int32(_C), nz[c % 3])

        def zrow(r, _, buf=buf):
            for j in range(_H // _L):
                buf[r, pl.ds(j * _L, _L)] = zvec
            return 0

        lax.fori_loop(lo, hi, zrow, 0)
        nz[c % 3] = k
        out_handles[c] = pltpu.async_copy(
            buf, out_hbm.at[pl.ds(row0 + c * _C, _C)], osems[c % 3])
    out_handles.pop(_NCH - 1).wait()


_expand = functools.partial(
    pl.kernel,
    out_type=jax.ShapeDtypeStruct((_B * _T, _H), jnp.float32),
    mesh=plsc.VectorSubcoreMesh(core_axis_name="c", subcore_axis_name="s"),
    compiler_params=pltpu.CompilerParams(
        needs_layout_passes=False, use_tc_tiling_on_sc=False),
    scratch_types=[
        pltpu.VMEM((_S,), jnp.float32),
        pltpu.VMEM((_S,), jnp.int32),
        pltpu.VMEM((_TPW,), jnp.int32),
        pltpu.VMEM((_L,), jnp.int32),
        pltpu.VMEM((_C, _H), jnp.float32),
        pltpu.VMEM((_C, _H), jnp.float32),
        pltpu.VMEM((_C, _H), jnp.float32),
        pltpu.SemaphoreType.DMA,
        pltpu.SemaphoreType.DMA,
        pltpu.SemaphoreType.DMA,
        pltpu.SemaphoreType.DMA,
        pltpu.SemaphoreType.DMA,
        pltpu.SemaphoreType.DMA,
    ],
)(_expand_body)


def kernel(x, durations, max_len):
    x2 = x.reshape(_B * _S, _H)
    dur2 = durations.reshape(_B * _S)
    ml = jnp.full((_L,), max_len, jnp.int32)
    out2 = _expand(x2, dur2, ml)
    return out2.reshape(_B, _T, _H)


# speedup vs baseline: 1.8939x; 1.8939x over previous
import functools

import jax
import jax.numpy as jnp
from jax import lax
from jax.experimental import pallas as pl
from jax.experimental.pallas import tpu as pltpu
from jax.experimental.pallas import tpu_sc as plsc

_B, _S, _H = 16, 512, 512
_T = 2048
_L = 16
_WPB = 2
_TPW = _T // _WPB
_C = 64
_NCH = _TPW // _C


def _round_half_even(v):
    ti = v.astype(jnp.int32)
    frac = v - ti.astype(jnp.float32)
    half = jnp.full((_L,), 0.5, jnp.float32)
    one = jnp.ones((_L,), jnp.int32)
    zero = jnp.zeros((_L,), jnp.int32)
    up = jnp.where(frac > half, one, zero)
    tie = jnp.where(frac == half, one, zero) & (ti & one)
    return ti + (up | tie)


def _expand_body(x_hbm, dur_hbm, ml_hbm, out_hbm,
                 dur_v, cum_v, idx_v, ml_v,
                 rows0_v, rows1_v, rows2_v,
                 gsem0, gsem1, gsem2, osem0, osem1, osem2):
    half = lax.axis_index("c")
    b = lax.axis_index("s")
    t0 = half * _TPW
    row0 = b * _T + t0

    pltpu.sync_copy(dur_hbm.at[pl.ds(b * _S, _S)], dur_v)
    pltpu.sync_copy(ml_hbm, ml_v)
    max_len = ml_v[...][0]

    def cs_body(i, carry):
        r = _round_half_even(dur_v[pl.ds(i * _L, _L)])
        cum = plsc.cumsum(r) + carry
        cum_v[pl.ds(i * _L, _L)] = cum
        return cum[_L - 1]

    total = lax.fori_loop(0, _S // _L, cs_body, jnp.int32(0))

    def ss_body(i, _):
        t = t0 + i * _L + lax.iota(jnp.int32, _L)
        base = jnp.zeros((_L,), jnp.int32)
        for step in (256, 128, 64, 32, 16, 8, 4, 2, 1):
            probe = base + (step - 1)
            g = plsc.load_gather(cum_v, [probe])
            base = jnp.where(g <= t, base + step, base)
        idx_v[pl.ds(i * _L, _L)] = base + b * _S
        return 0

    _PRE = 2 * _C // _L
    lax.fori_loop(0, _PRE, ss_body, 0)

    nv = jnp.clip(jnp.minimum(total, max_len) - t0, 0, _TPW)

    zvec = jnp.zeros((_L,), jnp.float32)
    bufs = (rows0_v, rows1_v, rows2_v)
    gsems = (gsem0, gsem1, gsem2)
    osems = (osem0, osem1, osem2)

    def chunk_k(c):
        return jnp.clip(nv - c * _C, 0, _C)

    def start_gather(c):
        @pl.when(chunk_k(c) > 0)
        def _():
            pltpu.async_copy(
                x_hbm.at[idx_v.at[pl.ds(c * _C, _C)]],
                bufs[c % 3], gsems[c % 3])

    def wait_gather(c):
        @pl.when(chunk_k(c) > 0)
        def _():
            pltpu.make_async_copy(
                x_hbm.at[pl.ds(0, _C)], bufs[c % 3], gsems[c % 3]).wait()

    out_handles = {}
    nz = [jnp.int32(_C)] * 3

    start_gather(0)
    start_gather(1)
    lax.fori_loop(_PRE, _TPW // _L, ss_body, 0)
    for c in range(_NCH):
        if c >= 1:
            out_handles.pop(c - 1).wait()
        if c + 2 < _NCH:
            start_gather(c + 2)
        wait_gather(c)
        k = chunk_k(c)
        buf = bufs[c % 3]
        lo = jnp.where(k > 0, k, jnp.int32(0))
        hi = jnp.where(k > 0, jnp.int32(_C), nz[c % 3])

        def zrow(r, _, buf=buf):
            for j in range(_H // _L):
                buf[r, pl.ds(j * _L, _L)] = zvec
            return 0

        lax.fori_loop(lo, hi, zrow, 0)
        nz[c % 3] = k
        out_handles[c] = pltpu.async_copy(
            buf, out_hbm.at[pl.ds(row0 + c * _C, _C)], osems[c % 3])
    out_handles.pop(_NCH - 1).wait()


_expand = functools.partial(
    pl.kernel,
    out_type=jax.ShapeDtypeStruct((_B * _T, _H), jnp.float32),
    mesh=plsc.VectorSubcoreMesh(core_axis_name="c", subcore_axis_name="s"),
    compiler_params=pltpu.CompilerParams(needs_layout_passes=False),
    scratch_types=[
        pltpu.VMEM((_S,), jnp.float32),
        pltpu.VMEM((_S,), jnp.int32),
        pltpu.VMEM((_TPW,), jnp.int32),
        pltpu.VMEM((_L,), jnp.int32),
        pltpu.VMEM((_C, _H), jnp.float32),
        pltpu.VMEM((_C, _H), jnp.float32),
        pltpu.VMEM((_C, _H), jnp.float32),
        pltpu.SemaphoreType.DMA,
        pltpu.SemaphoreType.DMA,
        pltpu.SemaphoreType.DMA,
        pltpu.SemaphoreType.DMA,
        pltpu.SemaphoreType.DMA,
        pltpu.SemaphoreType.DMA,
    ],
)(_expand_body)


def kernel(x, durations, max_len):
    x2 = x.reshape(_B * _S, _H)
    dur2 = durations.reshape(_B * _S)
    ml = jnp.full((_L,), max_len, jnp.int32)
    out2 = _expand(x2, dur2, ml)
    return out2.reshape(_B, _T, _H)
